# Initial kernel scaffold; baseline (speedup 1.0000x reference)
import jax
import jax.numpy as jnp
from jax.experimental import pallas as pl


def kernel(nodes, edges, senders, receivers, We1, be1, We2, be2, We3, be3, Wn1, bn1, Wn2, bn2, Wn3, bn3):
    raise NotImplementedError("write your pallas kernel here")


# speedup vs baseline: 3.0486x; 3.0486x over previous
import functools

import jax
import jax.numpy as jnp
from jax import lax
from jax.experimental import pallas as pl
from jax.experimental.pallas import tpu as pltpu
from jax.experimental.pallas import tpu_sc as plsc

_NC = 2
_NS = 16
_NW = _NC * _NS
_CH = 128


def _proj_body(nodes_ref, ws_ref, wr_ref, p_ref):
    x = nodes_ref[...]
    h = ws_ref.shape[1]
    p_ref[:, :h] = jnp.dot(x, ws_ref[...], preferred_element_type=jnp.float32)
    p_ref[:, h:] = jnp.dot(x, wr_ref[...], preferred_element_type=jnp.float32)


def _project(nodes, ws, wr):
    n, _ = nodes.shape
    h = ws.shape[1]
    return pl.pallas_call(
        _proj_body,
        out_shape=jax.ShapeDtypeStruct((n, 2 * h), jnp.float32),
    )(nodes, ws, wr)


def _make_sc_gather(n, e, h):
    epw = e // _NW
    assert e % _NW == 0
    nfull, tail = divmod(epw, _CH)
    assert tail % 8 == 0
    mesh = plsc.VectorSubcoreMesh(
        core_axis_name="c", subcore_axis_name="s",
        num_cores=_NC, num_subcores=_NS)

    @functools.partial(
        pl.kernel,
        out_type=jax.ShapeDtypeStruct((e, h), jnp.float32),
        mesh=mesh,
        scratch_types=[
            pltpu.VMEM((epw,), jnp.int32),
            pltpu.VMEM((epw,), jnp.int32),
            pltpu.VMEM((_CH, 2 * h), jnp.float32),
            pltpu.VMEM((_CH, 2 * h), jnp.float32),
            pltpu.VMEM((_CH, h), jnp.float32),
            pltpu.SemaphoreType.DMA,
            pltpu.SemaphoreType.DMA,
            pltpu.SemaphoreType.DMA,
        ],
    )
    def sc_gather(p_hbm, snd_hbm, rcv_hbm, g_hbm,
                  slab_s, slab_r, buf_s, buf_r, gbuf, sem_gs, sem_gr, sem_w):
        wid = lax.axis_index("s") * _NC + lax.axis_index("c")
        e0 = wid * epw
        pltpu.sync_copy(snd_hbm.at[pl.ds(e0, epw)], slab_s)
        pltpu.sync_copy(rcv_hbm.at[pl.ds(e0, epw)], slab_r)

        def chunk(off, cw):
            gs = pltpu.async_copy(p_hbm.at[slab_s.at[pl.ds(off, cw)]],
                                  buf_s.at[pl.ds(0, cw)], sem_gs)
            gr = pltpu.async_copy(p_hbm.at[slab_r.at[pl.ds(off, cw)]],
                                  buf_r.at[pl.ds(0, cw)], sem_gr)
            gs.wait()
            gr.wait()

            def row(r, _):
                for j in range(h // 16):
                    gbuf[r, pl.ds(j * 16, 16)] = (
                        buf_s[r, pl.ds(j * 16, 16)]
                        + buf_r[r, pl.ds(h + j * 16, 16)])
                return 0
            lax.fori_loop(0, cw, row, 0)
            w = pltpu.async_copy(gbuf.at[pl.ds(0, cw)],
                                 g_hbm.at[pl.ds(e0 + off, cw)], sem_w)
            w.wait()

        def body(i, _):
            chunk(i * _CH, _CH)
            return 0
        lax.fori_loop(0, nfull, body, 0)
        if tail:
            chunk(nfull * _CH, tail)

    return sc_gather


def _edge_mlp_body(g_ref, e_ref, we_ref, b1_ref, w2_ref, b2_ref,
                   w3_ref, b3_ref, out_ref):
    x = (g_ref[...]
         + jnp.dot(e_ref[...], we_ref[...], preferred_element_type=jnp.float32)
         + b1_ref[...])
    x = jax.nn.gelu(x)
    x = jax.nn.gelu(jnp.dot(x, w2_ref[...], preferred_element_type=jnp.float32)
                    + b2_ref[...])
    out_ref[...] = (jnp.dot(x, w3_ref[...], preferred_element_type=jnp.float32)
                    + b3_ref[...])


def _edge_mlp(g, edges, we, b1, w2, b2, w3, b3, rb=2000):
    e, ed = edges.shape
    h = we.shape[1]
    assert e % rb == 0
    grid = (e // rb,)
    full = lambda i: (0, 0)
    return pl.pallas_call(
        _edge_mlp_body,
        grid=grid,
        in_specs=[
            pl.BlockSpec((rb, h), lambda i: (i, 0)),
            pl.BlockSpec((rb, ed), lambda i: (i, 0)),
            pl.BlockSpec((ed, h), full),
            pl.BlockSpec((1, h), full),
            pl.BlockSpec((h, h), full),
            pl.BlockSpec((1, h), full),
            pl.BlockSpec((h, ed), full),
            pl.BlockSpec((1, ed), full),
        ],
        out_specs=pl.BlockSpec((rb, ed), lambda i: (i, 0)),
        out_shape=jax.ShapeDtypeStruct((e, ed), jnp.float32),
    )(g, edges, we, b1, w2, b2, w3, b3)


_W = 128


def _make_sc_scatter(n, e, ed):
    epw = e // _NW
    nfull, tail = divmod(epw, _CH)
    zr = n // _NS
    assert n % _NS == 0
    nzc = -(-zr // _CH)
    ztail = zr - (nzc - 1) * _CH
    mesh = plsc.VectorSubcoreMesh(
        core_axis_name="c", subcore_axis_name="s",
        num_cores=_NC, num_subcores=_NS)

    @functools.partial(
        pl.kernel,
        out_type=jax.ShapeDtypeStruct((_NC, _NS, zr, _W), jnp.float32),
        mesh=mesh,
        scratch_types=[
            pltpu.VMEM((_CH,), jnp.int32),
            pltpu.VMEM((_CH,), jnp.int32),
            pltpu.VMEM((_CH, ed), jnp.float32),
            pltpu.VMEM((_CH, _W), jnp.float32),
            pltpu.VMEM_SHARED((n, _W), jnp.float32),
            pltpu.SemaphoreType.DMA,
        ],
    )
    def sc_scatter(rcv_hbm, ne_hbm, out_hbm, idxb, ridx, ebuf16, ebuf,
                   table, sem):
        cid = lax.axis_index("c")
        sid = lax.axis_index("s")
        wid = sid * _NC + cid
        base = sid * zr
        e0 = wid * epw

        def fill_ramp(start):
            for j in range(_CH // 16):
                idxb[pl.ds(j * 16, 16)] = start + j * 16 + lax.iota(jnp.int32, 16)

        def clamp_ramp(limit, repl):
            for j in range(_CH // 16):
                v = idxb[pl.ds(j * 16, 16)]
                idxb[pl.ds(j * 16, 16)] = jnp.where(v < limit, v, repl)

        def zrow(i, _):
            for j in range(_W // 16):
                ebuf[i, pl.ds(j * 16, 16)] = jnp.zeros((16,), jnp.float32)
            return 0
        lax.fori_loop(0, _CH, zrow, 0)
        for c in range(nzc):
            fill_ramp(base + c * _CH)
            if c == nzc - 1:
                clamp_ramp(base + zr, base)
            pltpu.sync_copy(ebuf, table.at[idxb])
        plsc.subcore_barrier()

        def repack(i, _):
            ebuf[i, pl.ds(0, ed)] = ebuf16[i, pl.ds(0, ed)]
            return 0

        def body(i, _):
            off = e0 + i * _CH
            pltpu.sync_copy(rcv_hbm.at[pl.ds(off, _CH)], ridx)
            pltpu.sync_copy(ne_hbm.at[pl.ds(off, _CH)], ebuf16)
            lax.fori_loop(0, _CH, repack, 0)
            pltpu.sync_copy(ebuf, table.at[ridx], add=True)
            return 0
        lax.fori_loop(0, nfull, body, 0)
        if tail:
            off = e0 + nfull * _CH
            for j in range(_CH // 16):
                ridx[pl.ds(j * 16, 16)] = jnp.zeros((16,), jnp.int32)
            def zpad(i, _):
                ebuf[i, pl.ds(0, ed)] = jnp.zeros((ed,), jnp.float32)
                return 0
            lax.fori_loop(tail, _CH, zpad, 0)
            pltpu.sync_copy(rcv_hbm.at[pl.ds(off, tail)], ridx.at[pl.ds(0, tail)])
            pltpu.sync_copy(ne_hbm.at[pl.ds(off, tail)], ebuf16.at[pl.ds(0, tail)])
            lax.fori_loop(0, tail, repack, 0)
            pltpu.sync_copy(ebuf, table.at[ridx], add=True)
        plsc.subcore_barrier()

        for c in range(nzc):
            rows = _CH if c < nzc - 1 else ztail
            fill_ramp(base + c * _CH)
            pltpu.async_copy(table.at[idxb.at[pl.ds(0, rows)]],
                             ebuf.at[pl.ds(0, rows)], sem).wait()
            pltpu.sync_copy(ebuf.at[pl.ds(0, rows)],
                            out_hbm.at[cid, sid, pl.ds(c * _CH, rows)])

    return sc_scatter


def _node_mlp_body(nodes_ref, agg_ref, wn_ref, wa_ref, b1_ref, w2_ref, b2_ref,
                   w3_ref, b3_ref, out_ref):
    agg = agg_ref[0] + agg_ref[1]
    x = (jnp.dot(nodes_ref[...], wn_ref[...], preferred_element_type=jnp.float32)
         + jnp.dot(agg, wa_ref[...], preferred_element_type=jnp.float32)
         + b1_ref[...])
    x = jax.nn.gelu(x)
    x = jax.nn.gelu(jnp.dot(x, w2_ref[...], preferred_element_type=jnp.float32)
                    + b2_ref[...])
    out_ref[...] = (jnp.dot(x, w3_ref[...], preferred_element_type=jnp.float32)
                    + b3_ref[...])


def _node_mlp(nodes, agg2, wn, wa, b1, w2, b2, w3, b3):
    n, nd = nodes.shape
    return pl.pallas_call(
        _node_mlp_body,
        out_shape=jax.ShapeDtypeStruct((n, nd), jnp.float32),
    )(nodes, agg2, wn, wa, b1, w2, b2, w3, b3)


def kernel(nodes, edges, senders, receivers,
           We1, be1, We2, be2, We3, be3,
           Wn1, bn1, Wn2, bn2, Wn3, bn3):
    n, nd = nodes.shape
    e, ed = edges.shape
    h = We1.shape[1]
    senders = senders.astype(jnp.int32)
    receivers = receivers.astype(jnp.int32)

    ws, wr, we = We1[:nd], We1[nd:2 * nd], We1[2 * nd:]
    p = _project(nodes, ws, wr)

    g = _make_sc_gather(n, e, h)(p, senders, receivers)

    new_edges = _edge_mlp(
        g, edges, we,
        be1.reshape(1, h), We2, be2.reshape(1, h), We3, be3.reshape(1, ed))

    agg2 = _make_sc_scatter(n, e, ed)(receivers, new_edges)
    agg2 = agg2.reshape(_NC, n, _W)[:, :, :ed]

    new_nodes = _node_mlp(
        nodes, agg2, Wn1[:nd], Wn1[nd:],
        bn1.reshape(1, h), Wn2, bn2.reshape(1, h), Wn3, bn3.reshape(1, nd))
    return new_nodes, new_edges


# speedup vs baseline: 3.6175x; 1.1866x over previous
import functools

import jax
import jax.numpy as jnp
from jax import lax
from jax.experimental import pallas as pl
from jax.experimental.pallas import tpu as pltpu
from jax.experimental.pallas import tpu_sc as plsc

_NC = 2
_NS = 16
_NW = _NC * _NS
_CH = 128


def _proj_body(nodes_ref, ws_ref, wr_ref, p_ref):
    x = nodes_ref[...]
    h = ws_ref.shape[1]
    p_ref[:, :h] = jnp.dot(x, ws_ref[...], preferred_element_type=jnp.float32)
    p_ref[:, h:] = jnp.dot(x, wr_ref[...], preferred_element_type=jnp.float32)


def _project(nodes, ws, wr):
    n, _ = nodes.shape
    h = ws.shape[1]
    return pl.pallas_call(
        _proj_body,
        out_shape=jax.ShapeDtypeStruct((n, 2 * h), jnp.float32),
    )(nodes, ws, wr)


def _make_sc_gather(n, e, h):
    epw = e // _NW
    assert e % _NW == 0
    nfull, tail = divmod(epw, _CH)
    assert tail % 8 == 0
    mesh = plsc.VectorSubcoreMesh(
        core_axis_name="c", subcore_axis_name="s",
        num_cores=_NC, num_subcores=_NS)

    npairs = nfull // 2
    assert nfull % 2 == 0 and npairs >= 2

    @functools.partial(
        pl.kernel,
        out_type=jax.ShapeDtypeStruct((e, h), jnp.float32),
        mesh=mesh,
        scratch_types=[
            pltpu.VMEM((epw,), jnp.int32),
            pltpu.VMEM((epw,), jnp.int32),
            pltpu.VMEM((2, _CH, 2 * h), jnp.float32),
            pltpu.VMEM((2, _CH, 2 * h), jnp.float32),
            pltpu.VMEM((2, _CH, h), jnp.float32),
            pltpu.SemaphoreType.DMA,
            pltpu.SemaphoreType.DMA,
            pltpu.SemaphoreType.DMA,
            pltpu.SemaphoreType.DMA,
            pltpu.SemaphoreType.DMA,
            pltpu.SemaphoreType.DMA,
        ],
    )
    def sc_gather(p_hbm, snd_hbm, rcv_hbm, g_hbm,
                  slab_s, slab_r, buf_s, buf_r, gbuf,
                  sgs0, sgr0, sgs1, sgr1, sw0, sw1):
        wid = lax.axis_index("s") * _NC + lax.axis_index("c")
        e0 = wid * epw
        pltpu.sync_copy(snd_hbm.at[pl.ds(e0, epw)], slab_s)
        pltpu.sync_copy(rcv_hbm.at[pl.ds(e0, epw)], slab_r)
        sg = ((sgs0, sgr0), (sgs1, sgr1))
        sw = (sw0, sw1)

        def g_issue(off, p):
            pltpu.async_copy(p_hbm.at[slab_s.at[pl.ds(off, _CH)]],
                             buf_s.at[p], sg[p][0])
            pltpu.async_copy(p_hbm.at[slab_r.at[pl.ds(off, _CH)]],
                             buf_r.at[p], sg[p][1])

        def g_wait(p):
            pltpu.make_async_copy(p_hbm.at[slab_s.at[pl.ds(0, _CH)]],
                                  buf_s.at[p], sg[p][0]).wait()
            pltpu.make_async_copy(p_hbm.at[slab_r.at[pl.ds(0, _CH)]],
                                  buf_r.at[p], sg[p][1]).wait()

        def valu(p):
            def row(r, _):
                for j in range(h // 16):
                    gbuf[p, r, pl.ds(j * 16, 16)] = (
                        buf_s[p, r, pl.ds(j * 16, 16)]
                        + buf_r[p, r, pl.ds(h + j * 16, 16)])
                return 0
            lax.fori_loop(0, _CH, row, 0)

        def w_issue(off, p):
            pltpu.async_copy(gbuf.at[p], g_hbm.at[pl.ds(e0 + off, _CH)], sw[p])

        def w_wait(p):
            pltpu.make_async_copy(gbuf.at[p], g_hbm.at[pl.ds(e0, _CH)],
                                  sw[p]).wait()

        g_issue(0, 0)
        g_issue(_CH, 1)
        g_wait(0)
        valu(0)
        w_issue(0, 0)
        g_issue(2 * _CH, 0)
        g_wait(1)
        valu(1)
        w_issue(_CH, 1)
        g_issue(3 * _CH, 1)

        def body(k, _):
            c0 = 2 * k
            for p in range(2):
                c = c0 + p
                g_wait(p)
                w_wait(p)
                valu(p)
                w_issue(c * _CH, p)
                @pl.when(c + 2 < nfull)
                def _():
                    g_issue((c + 2) * _CH, p)
            return 0
        lax.fori_loop(1, npairs, body, 0)
        w_wait(0)
        w_wait(1)

        if tail:
            off = nfull * _CH
            gs = pltpu.async_copy(p_hbm.at[slab_s.at[pl.ds(off, tail)]],
                                  buf_s.at[0, pl.ds(0, tail)], sgs0)
            gr = pltpu.async_copy(p_hbm.at[slab_r.at[pl.ds(off, tail)]],
                                  buf_r.at[0, pl.ds(0, tail)], sgr0)
            gs.wait()
            gr.wait()

            def trow(r, _):
                for j in range(h // 16):
                    gbuf[0, r, pl.ds(j * 16, 16)] = (
                        buf_s[0, r, pl.ds(j * 16, 16)]
                        + buf_r[0, r, pl.ds(h + j * 16, 16)])
                return 0
            lax.fori_loop(0, tail, trow, 0)
            pltpu.async_copy(gbuf.at[0, pl.ds(0, tail)],
                             g_hbm.at[pl.ds(e0 + off, tail)], sw0).wait()

    return sc_gather


def _edge_mlp_body(g_ref, e_ref, we_ref, b1_ref, w2_ref, b2_ref,
                   w3_ref, b3_ref, out_ref):
    x = (g_ref[...]
         + jnp.dot(e_ref[...], we_ref[...], preferred_element_type=jnp.float32)
         + b1_ref[...])
    x = jax.nn.gelu(x)
    x = jax.nn.gelu(jnp.dot(x, w2_ref[...], preferred_element_type=jnp.float32)
                    + b2_ref[...])
    out_ref[...] = (jnp.dot(x, w3_ref[...], preferred_element_type=jnp.float32)
                    + b3_ref[...])


def _edge_mlp(g, edges, we, b1, w2, b2, w3, b3, rb=2000):
    e, ed = edges.shape
    h = we.shape[1]
    assert e % rb == 0
    grid = (e // rb,)
    full = lambda i: (0, 0)
    return pl.pallas_call(
        _edge_mlp_body,
        grid=grid,
        in_specs=[
            pl.BlockSpec((rb, h), lambda i: (i, 0)),
            pl.BlockSpec((rb, ed), lambda i: (i, 0)),
            pl.BlockSpec((ed, h), full),
            pl.BlockSpec((1, h), full),
            pl.BlockSpec((h, h), full),
            pl.BlockSpec((1, h), full),
            pl.BlockSpec((h, ed), full),
            pl.BlockSpec((1, ed), full),
        ],
        out_specs=pl.BlockSpec((rb, ed), lambda i: (i, 0)),
        out_shape=jax.ShapeDtypeStruct((e, ed), jnp.float32),
    )(g, edges, we, b1, w2, b2, w3, b3)


_W = 128


def _make_sc_scatter(n, e, ed):
    sch = 64
    epw = e // _NW
    nfull, tail = divmod(epw, sch)
    zr = n // _NS
    assert n % _NS == 0
    nzc = -(-zr // sch)
    ztail = zr - (nzc - 1) * sch
    mesh = plsc.VectorSubcoreMesh(
        core_axis_name="c", subcore_axis_name="s",
        num_cores=_NC, num_subcores=_NS)

    npairs = nfull // 2
    assert nfull % 2 == 0 and npairs >= 2

    @functools.partial(
        pl.kernel,
        out_type=jax.ShapeDtypeStruct((_NC, _NS, zr, _W), jnp.float32),
        mesh=mesh,
        scratch_types=[
            pltpu.VMEM((sch,), jnp.int32),
            pltpu.VMEM((2, sch), jnp.int32),
            pltpu.VMEM((2, sch, ed), jnp.float32),
            pltpu.VMEM((2, sch, _W), jnp.float32),
            pltpu.VMEM_SHARED((n, _W), jnp.float32),
            pltpu.SemaphoreType.DMA,
            pltpu.SemaphoreType.DMA,
            pltpu.SemaphoreType.DMA,
            pltpu.SemaphoreType.DMA,
            pltpu.SemaphoreType.DMA,
        ],
    )
    def sc_scatter(rcv_hbm, ne_hbm, out_hbm, idxb, ridx, ebuf16, ebuf,
                   table, sem, sl0, sl1, ss0, ss1):
        cid = lax.axis_index("c")
        sid = lax.axis_index("s")
        wid = sid * _NC + cid
        base = sid * zr
        e0 = wid * epw
        sl = (sl0, sl1)
        ss = (ss0, ss1)

        def fill_ramp(start):
            for j in range(sch // 16):
                idxb[pl.ds(j * 16, 16)] = start + j * 16 + lax.iota(jnp.int32, 16)

        def clamp_ramp(limit, repl):
            for j in range(sch // 16):
                v = idxb[pl.ds(j * 16, 16)]
                idxb[pl.ds(j * 16, 16)] = jnp.where(v < limit, v, repl)

        def zrow(i, _):
            for j in range(_W // 16):
                ebuf[0, i, pl.ds(j * 16, 16)] = jnp.zeros((16,), jnp.float32)
                ebuf[1, i, pl.ds(j * 16, 16)] = jnp.zeros((16,), jnp.float32)
            return 0
        lax.fori_loop(0, sch, zrow, 0)
        for c in range(nzc):
            fill_ramp(base + c * sch)
            if c == nzc - 1:
                clamp_ramp(base + zr, base)
            pltpu.sync_copy(ebuf.at[0], table.at[idxb])
        plsc.subcore_barrier()

        def l_issue(off, p):
            pltpu.async_copy(rcv_hbm.at[pl.ds(e0 + off, sch)], ridx.at[p], sl[p])
            pltpu.async_copy(ne_hbm.at[pl.ds(e0 + off, sch)], ebuf16.at[p], sl[p])

        def l_wait(p):
            pltpu.make_async_copy(rcv_hbm.at[pl.ds(e0, sch)], ridx.at[p],
                                  sl[p]).wait()
            pltpu.make_async_copy(ne_hbm.at[pl.ds(e0, sch)], ebuf16.at[p],
                                  sl[p]).wait()

        def repack(p):
            def rrow(i, _):
                ebuf[p, i, pl.ds(0, ed)] = ebuf16[p, i, pl.ds(0, ed)]
                return 0
            lax.fori_loop(0, sch, rrow, 0)

        def s_issue(p):
            pltpu.async_copy(ebuf.at[p], table.at[ridx.at[p]], ss[p], add=True)

        def s_wait(p):
            pltpu.make_async_copy(ebuf.at[p], table.at[ridx.at[p]],
                                  ss[p]).wait()

        l_issue(0, 0)
        l_issue(sch, 1)
        for p in range(2):
            l_wait(p)
            repack(p)
            s_issue(p)
        for p in range(2):
            s_wait(p)
            l_issue((2 + p) * sch, p)

        def body(k, _):
            c0 = 2 * k
            for p in range(2):
                l_wait(p)
                repack(p)
                s_issue(p)
            for p in range(2):
                c = c0 + p
                s_wait(p)
                @pl.when(c + 2 < nfull)
                def _():
                    l_issue((c + 2) * sch, p)
            return 0
        lax.fori_loop(1, npairs, body, 0)

        if tail:
            off = e0 + nfull * sch
            for j in range(sch // 16):
                ridx[0, pl.ds(j * 16, 16)] = jnp.zeros((16,), jnp.int32)
            def zpad(i, _):
                ebuf[0, i, pl.ds(0, ed)] = jnp.zeros((ed,), jnp.float32)
                return 0
            lax.fori_loop(tail, sch, zpad, 0)
            pltpu.sync_copy(rcv_hbm.at[pl.ds(off, tail)],
                            ridx.at[0, pl.ds(0, tail)])
            pltpu.sync_copy(ne_hbm.at[pl.ds(off, tail)],
                            ebuf16.at[0, pl.ds(0, tail)])
            def trow(i, _):
                ebuf[0, i, pl.ds(0, ed)] = ebuf16[0, i, pl.ds(0, ed)]
                return 0
            lax.fori_loop(0, tail, trow, 0)
            pltpu.sync_copy(ebuf.at[0], table.at[ridx.at[0]], add=True)
        plsc.subcore_barrier()

        for c in range(nzc):
            rows = sch if c < nzc - 1 else ztail
            fill_ramp(base + c * sch)
            pltpu.async_copy(table.at[idxb.at[pl.ds(0, rows)]],
                             ebuf.at[0, pl.ds(0, rows)], sem).wait()
            pltpu.sync_copy(ebuf.at[0, pl.ds(0, rows)],
                            out_hbm.at[cid, sid, pl.ds(c * sch, rows)])

    return sc_scatter


def _node_mlp_body(nodes_ref, agg_ref, wn_ref, wa_ref, b1_ref, w2_ref, b2_ref,
                   w3_ref, b3_ref, out_ref):
    agg = agg_ref[0] + agg_ref[1]
    x = (jnp.dot(nodes_ref[...], wn_ref[...], preferred_element_type=jnp.float32)
         + jnp.dot(agg, wa_ref[...], preferred_element_type=jnp.float32)
         + b1_ref[...])
    x = jax.nn.gelu(x)
    x = jax.nn.gelu(jnp.dot(x, w2_ref[...], preferred_element_type=jnp.float32)
                    + b2_ref[...])
    out_ref[...] = (jnp.dot(x, w3_ref[...], preferred_element_type=jnp.float32)
                    + b3_ref[...])


def _node_mlp(nodes, agg2, wn, wa, b1, w2, b2, w3, b3):
    n, nd = nodes.shape
    return pl.pallas_call(
        _node_mlp_body,
        out_shape=jax.ShapeDtypeStruct((n, nd), jnp.float32),
    )(nodes, agg2, wn, wa, b1, w2, b2, w3, b3)


def kernel(nodes, edges, senders, receivers,
           We1, be1, We2, be2, We3, be3,
           Wn1, bn1, Wn2, bn2, Wn3, bn3):
    n, nd = nodes.shape
    e, ed = edges.shape
    h = We1.shape[1]
    senders = senders.astype(jnp.int32)
    receivers = receivers.astype(jnp.int32)

    ws, wr, we = We1[:nd], We1[nd:2 * nd], We1[2 * nd:]
    p = _project(nodes, ws, wr)

    g = _make_sc_gather(n, e, h)(p, senders, receivers)

    new_edges = _edge_mlp(
        g, edges, we,
        be1.reshape(1, h), We2, be2.reshape(1, h), We3, be3.reshape(1, ed))

    agg2 = _make_sc_scatter(n, e, ed)(receivers, new_edges)
    agg2 = agg2.reshape(_NC, n, _W)[:, :, :ed]

    new_nodes = _node_mlp(
        nodes, agg2, Wn1[:nd], Wn1[nd:],
        bn1.reshape(1, h), Wn2, bn2.reshape(1, h), Wn3, bn3.reshape(1, nd))
    return new_nodes, new_edges


# speedup vs baseline: 4.1153x; 1.1376x over previous
import functools

import jax
import jax.numpy as jnp
from jax import lax
from jax.experimental import pallas as pl
from jax.experimental.pallas import tpu as pltpu
from jax.experimental.pallas import tpu_sc as plsc

_NC = 2
_NS = 16
_NW = _NC * _NS
_CH = 128


def _proj_body(nodes_ref, ws_ref, wr_ref, p_ref):
    x = nodes_ref[...]
    h = ws_ref.shape[1]
    p_ref[:, :h] = jnp.dot(x, ws_ref[...], preferred_element_type=jnp.float32)
    p_ref[:, h:] = jnp.dot(x, wr_ref[...], preferred_element_type=jnp.float32)


def _project(nodes, ws, wr):
    n, _ = nodes.shape
    h = ws.shape[1]
    return pl.pallas_call(
        _proj_body,
        out_shape=jax.ShapeDtypeStruct((n, 2 * h), jnp.float32),
    )(nodes, ws, wr)


def _make_sc_gather(n, e, h):
    epw = e // _NW
    assert e % _NW == 0
    nfull, tail = divmod(epw, _CH)
    assert tail % 8 == 0
    mesh = plsc.VectorSubcoreMesh(
        core_axis_name="c", subcore_axis_name="s",
        num_cores=_NC, num_subcores=_NS)

    npairs = nfull // 2
    assert nfull % 2 == 0 and npairs >= 2

    @functools.partial(
        pl.kernel,
        out_type=jax.ShapeDtypeStruct((e, h), jnp.float32),
        mesh=mesh,
        scratch_types=[
            pltpu.VMEM((epw,), jnp.int32),
            pltpu.VMEM((epw,), jnp.int32),
            pltpu.VMEM((2, _CH, 2 * h), jnp.float32),
            pltpu.VMEM((2, _CH, 2 * h), jnp.float32),
            pltpu.VMEM((2, _CH, h), jnp.float32),
            pltpu.SemaphoreType.DMA,
            pltpu.SemaphoreType.DMA,
            pltpu.SemaphoreType.DMA,
            pltpu.SemaphoreType.DMA,
            pltpu.SemaphoreType.DMA,
            pltpu.SemaphoreType.DMA,
        ],
    )
    def sc_gather(p_hbm, snd_hbm, rcv_hbm, g_hbm,
                  slab_s, slab_r, buf_s, buf_r, gbuf,
                  sgs0, sgr0, sgs1, sgr1, sw0, sw1):
        wid = lax.axis_index("s") * _NC + lax.axis_index("c")
        e0 = wid * epw
        pltpu.sync_copy(snd_hbm.at[pl.ds(e0, epw)], slab_s)
        pltpu.sync_copy(rcv_hbm.at[pl.ds(e0, epw)], slab_r)
        sg = ((sgs0, sgr0), (sgs1, sgr1))
        sw = (sw0, sw1)

        def g_issue(off, p):
            pltpu.async_copy(p_hbm.at[slab_s.at[pl.ds(off, _CH)]],
                             buf_s.at[p], sg[p][0])
            pltpu.async_copy(p_hbm.at[slab_r.at[pl.ds(off, _CH)]],
                             buf_r.at[p], sg[p][1])

        def g_wait(p):
            pltpu.make_async_copy(p_hbm.at[slab_s.at[pl.ds(0, _CH)]],
                                  buf_s.at[p], sg[p][0]).wait()
            pltpu.make_async_copy(p_hbm.at[slab_r.at[pl.ds(0, _CH)]],
                                  buf_r.at[p], sg[p][1]).wait()

        def valu(p):
            def row(r, _):
                for j in range(h // 16):
                    gbuf[p, r, pl.ds(j * 16, 16)] = (
                        buf_s[p, r, pl.ds(j * 16, 16)]
                        + buf_r[p, r, pl.ds(h + j * 16, 16)])
                return 0
            lax.fori_loop(0, _CH, row, 0)

        def w_issue(off, p):
            pltpu.async_copy(gbuf.at[p], g_hbm.at[pl.ds(e0 + off, _CH)], sw[p])

        def w_wait(p):
            pltpu.make_async_copy(gbuf.at[p], g_hbm.at[pl.ds(e0, _CH)],
                                  sw[p]).wait()

        g_issue(0, 0)
        g_issue(_CH, 1)
        g_wait(0)
        valu(0)
        w_issue(0, 0)
        g_issue(2 * _CH, 0)
        g_wait(1)
        valu(1)
        w_issue(_CH, 1)
        g_issue(3 * _CH, 1)

        def body(k, _):
            c0 = 2 * k
            for p in range(2):
                c = c0 + p
                g_wait(p)
                w_wait(p)
                valu(p)
                w_issue(c * _CH, p)
                @pl.when(c + 2 < nfull)
                def _():
                    g_issue((c + 2) * _CH, p)
            return 0
        lax.fori_loop(1, npairs, body, 0)
        w_wait(0)
        w_wait(1)

        if tail:
            off = nfull * _CH
            gs = pltpu.async_copy(p_hbm.at[slab_s.at[pl.ds(off, tail)]],
                                  buf_s.at[0, pl.ds(0, tail)], sgs0)
            gr = pltpu.async_copy(p_hbm.at[slab_r.at[pl.ds(off, tail)]],
                                  buf_r.at[0, pl.ds(0, tail)], sgr0)
            gs.wait()
            gr.wait()

            def trow(r, _):
                for j in range(h // 16):
                    gbuf[0, r, pl.ds(j * 16, 16)] = (
                        buf_s[0, r, pl.ds(j * 16, 16)]
                        + buf_r[0, r, pl.ds(h + j * 16, 16)])
                return 0
            lax.fori_loop(0, tail, trow, 0)
            pltpu.async_copy(gbuf.at[0, pl.ds(0, tail)],
                             g_hbm.at[pl.ds(e0 + off, tail)], sw0).wait()

    return sc_gather


def _edge_mlp_body(g_ref, e_ref, we_ref, b1_ref, w2_ref, b2_ref,
                   w3_ref, b3_ref, out_ref):
    x = (g_ref[...]
         + jnp.dot(e_ref[...], we_ref[...], preferred_element_type=jnp.float32)
         + b1_ref[...])
    x = jax.nn.gelu(x)
    x = jax.nn.gelu(jnp.dot(x, w2_ref[...], preferred_element_type=jnp.float32)
                    + b2_ref[...])
    out_ref[...] = (jnp.dot(x, w3_ref[...], preferred_element_type=jnp.float32)
                    + b3_ref[...])


def _edge_mlp(g, edges, we, b1, w2, b2, w3, b3, rb=8000):
    e, ed = edges.shape
    h = we.shape[1]
    assert e % rb == 0
    grid = (e // rb,)
    full = lambda i: (0, 0)
    return pl.pallas_call(
        _edge_mlp_body,
        grid=grid,
        in_specs=[
            pl.BlockSpec((rb, h), lambda i: (i, 0)),
            pl.BlockSpec((rb, ed), lambda i: (i, 0)),
            pl.BlockSpec((ed, h), full),
            pl.BlockSpec((1, h), full),
            pl.BlockSpec((h, h), full),
            pl.BlockSpec((1, h), full),
            pl.BlockSpec((h, ed), full),
            pl.BlockSpec((1, ed), full),
        ],
        out_specs=pl.BlockSpec((rb, ed), lambda i: (i, 0)),
        out_shape=jax.ShapeDtypeStruct((e, ed), jnp.float32),
    )(g, edges, we, b1, w2, b2, w3, b3)


_W = 128


def _make_sc_scatter(n, e, ed):
    sch = 64
    epw = e // _NW
    nfull, tail = divmod(epw, sch)
    zr = n // _NS
    assert n % _NS == 0
    nzc = -(-zr // sch)
    ztail = zr - (nzc - 1) * sch
    mesh = plsc.VectorSubcoreMesh(
        core_axis_name="c", subcore_axis_name="s",
        num_cores=_NC, num_subcores=_NS)

    npairs = nfull // 2
    assert nfull % 2 == 0 and npairs >= 2

    @functools.partial(
        pl.kernel,
        out_type=jax.ShapeDtypeStruct((_NC, _NS, zr, _W), jnp.float32),
        mesh=mesh,
        scratch_types=[
            pltpu.VMEM((sch,), jnp.int32),
            pltpu.VMEM((2, sch), jnp.int32),
            pltpu.VMEM((2, sch, ed), jnp.float32),
            pltpu.VMEM((2, sch, _W), jnp.float32),
            pltpu.VMEM_SHARED((n, _W), jnp.float32),
            pltpu.SemaphoreType.DMA,
            pltpu.SemaphoreType.DMA,
            pltpu.SemaphoreType.DMA,
            pltpu.SemaphoreType.DMA,
            pltpu.SemaphoreType.DMA,
        ],
    )
    def sc_scatter(rcv_hbm, ne_hbm, out_hbm, idxb, ridx, ebuf16, ebuf,
                   table, sem, sl0, sl1, ss0, ss1):
        cid = lax.axis_index("c")
        sid = lax.axis_index("s")
        wid = sid * _NC + cid
        base = sid * zr
        e0 = wid * epw
        sl = (sl0, sl1)
        ss = (ss0, ss1)

        def fill_ramp(start):
            for j in range(sch // 16):
                idxb[pl.ds(j * 16, 16)] = start + j * 16 + lax.iota(jnp.int32, 16)

        def clamp_ramp(limit, repl):
            for j in range(sch // 16):
                v = idxb[pl.ds(j * 16, 16)]
                idxb[pl.ds(j * 16, 16)] = jnp.where(v < limit, v, repl)

        def zrow(i, _):
            for j in range(_W // 16):
                ebuf[0, i, pl.ds(j * 16, 16)] = jnp.zeros((16,), jnp.float32)
                ebuf[1, i, pl.ds(j * 16, 16)] = jnp.zeros((16,), jnp.float32)
            return 0
        lax.fori_loop(0, sch, zrow, 0)
        for c in range(nzc):
            fill_ramp(base + c * sch)
            if c == nzc - 1:
                clamp_ramp(base + zr, base)
            pltpu.sync_copy(ebuf.at[0], table.at[idxb])
        plsc.subcore_barrier()

        def l_issue(off, p):
            pltpu.async_copy(rcv_hbm.at[pl.ds(e0 + off, sch)], ridx.at[p], sl[p])
            pltpu.async_copy(ne_hbm.at[pl.ds(e0 + off, sch)], ebuf16.at[p], sl[p])

        def l_wait(p):
            pltpu.make_async_copy(rcv_hbm.at[pl.ds(e0, sch)], ridx.at[p],
                                  sl[p]).wait()
            pltpu.make_async_copy(ne_hbm.at[pl.ds(e0, sch)], ebuf16.at[p],
                                  sl[p]).wait()

        def repack(p):
            def rrow(i, _):
                ebuf[p, i, pl.ds(0, ed)] = ebuf16[p, i, pl.ds(0, ed)]
                return 0
            lax.fori_loop(0, sch, rrow, 0)

        def s_issue(p):
            pltpu.async_copy(ebuf.at[p], table.at[ridx.at[p]], ss[p], add=True)

        def s_wait(p):
            pltpu.make_async_copy(ebuf.at[p], table.at[ridx.at[p]],
                                  ss[p]).wait()

        l_issue(0, 0)
        l_issue(sch, 1)
        for p in range(2):
            l_wait(p)
            repack(p)
            s_issue(p)
        for p in range(2):
            s_wait(p)
            l_issue((2 + p) * sch, p)

        def body(k, _):
            c0 = 2 * k
            for p in range(2):
                l_wait(p)
                repack(p)
                s_issue(p)
            for p in range(2):
                c = c0 + p
                s_wait(p)
                @pl.when(c + 2 < nfull)
                def _():
                    l_issue((c + 2) * sch, p)
            return 0
        lax.fori_loop(1, npairs, body, 0)

        if tail:
            off = e0 + nfull * sch
            for j in range(sch // 16):
                ridx[0, pl.ds(j * 16, 16)] = jnp.zeros((16,), jnp.int32)
            def zpad(i, _):
                ebuf[0, i, pl.ds(0, ed)] = jnp.zeros((ed,), jnp.float32)
                return 0
            lax.fori_loop(tail, sch, zpad, 0)
            pltpu.sync_copy(rcv_hbm.at[pl.ds(off, tail)],
                            ridx.at[0, pl.ds(0, tail)])
            pltpu.sync_copy(ne_hbm.at[pl.ds(off, tail)],
                            ebuf16.at[0, pl.ds(0, tail)])
            def trow(i, _):
                ebuf[0, i, pl.ds(0, ed)] = ebuf16[0, i, pl.ds(0, ed)]
                return 0
            lax.fori_loop(0, tail, trow, 0)
            pltpu.sync_copy(ebuf.at[0], table.at[ridx.at[0]], add=True)
        plsc.subcore_barrier()

        for c in range(nzc):
            rows = sch if c < nzc - 1 else ztail
            fill_ramp(base + c * sch)
            pltpu.async_copy(table.at[idxb.at[pl.ds(0, rows)]],
                             ebuf.at[0, pl.ds(0, rows)], sem).wait()
            pltpu.sync_copy(ebuf.at[0, pl.ds(0, rows)],
                            out_hbm.at[cid, sid, pl.ds(c * sch, rows)])

    return sc_scatter


def _node_mlp_body(nodes_ref, agg_ref, wn_ref, wa_ref, b1_ref, w2_ref, b2_ref,
                   w3_ref, b3_ref, out_ref):
    agg = agg_ref[0] + agg_ref[1]
    x = (jnp.dot(nodes_ref[...], wn_ref[...], preferred_element_type=jnp.float32)
         + jnp.dot(agg, wa_ref[...], preferred_element_type=jnp.float32)
         + b1_ref[...])
    x = jax.nn.gelu(x)
    x = jax.nn.gelu(jnp.dot(x, w2_ref[...], preferred_element_type=jnp.float32)
                    + b2_ref[...])
    out_ref[...] = (jnp.dot(x, w3_ref[...], preferred_element_type=jnp.float32)
                    + b3_ref[...])


def _node_mlp(nodes, agg2, wn, wa, b1, w2, b2, w3, b3):
    n, nd = nodes.shape
    return pl.pallas_call(
        _node_mlp_body,
        out_shape=jax.ShapeDtypeStruct((n, nd), jnp.float32),
    )(nodes, agg2, wn, wa, b1, w2, b2, w3, b3)


def kernel(nodes, edges, senders, receivers,
           We1, be1, We2, be2, We3, be3,
           Wn1, bn1, Wn2, bn2, Wn3, bn3):
    n, nd = nodes.shape
    e, ed = edges.shape
    h = We1.shape[1]
    senders = senders.astype(jnp.int32)
    receivers = receivers.astype(jnp.int32)

    ws, wr, we = We1[:nd], We1[nd:2 * nd], We1[2 * nd:]
    p = _project(nodes, ws, wr)

    g = _make_sc_gather(n, e, h)(p, senders, receivers)

    new_edges = _edge_mlp(
        g, edges, we,
        be1.reshape(1, h), We2, be2.reshape(1, h), We3, be3.reshape(1, ed))

    agg2 = _make_sc_scatter(n, e, ed)(receivers, new_edges)
    agg2 = agg2.reshape(_NC, n, _W)[:, :, :ed]

    new_nodes = _node_mlp(
        nodes, agg2, Wn1[:nd], Wn1[nd:],
        bn1.reshape(1, h), Wn2, bn2.reshape(1, h), Wn3, bn3.reshape(1, nd))
    return new_nodes, new_edges
